# baseline (speedup 1.0000x reference)
import functools

import jax
import jax.numpy as jnp
from jax import lax
from jax.experimental import pallas as pl
from jax.experimental.pallas import tpu as pltpu
from jax.experimental.pallas import tpu_sc as plsc

_L16 = 16


@functools.lru_cache(maxsize=None)
def _make_sc_kernel(N, D, CH, POS2):
    NW = 32
    rows_per_w = N // NW
    n_chunks = rows_per_w // CH
    mesh = plsc.VectorSubcoreMesh(core_axis_name="c", subcore_axis_name="s")

    @functools.partial(
        pl.kernel,
        out_type=jax.ShapeDtypeStruct((N, D), jnp.float32),
        mesh=mesh,
        scratch_types=[
            pltpu.VMEM((CH,), jnp.int32),
            pltpu.VMEM((CH,), jnp.int32),
            pltpu.VMEM((CH,), jnp.int32),
            pltpu.VMEM((CH, D), jnp.float32),
            pltpu.VMEM((CH, D), jnp.float32),
            pltpu.VMEM((POS2, D), jnp.float32),
            pltpu.SemaphoreType.DMA,
        ],
    )
    def sc_kernel(tokens_hbm, segs_hbm, tok_table_hbm, aux_hbm, pos2_hbm,
                  out_hbm, tok_v, seg_v, cidx_v, acc_v, aux_v, pos_v, sem):
        wid = lax.axis_index("s") * 2 + lax.axis_index("c")
        w_base = wid * rows_per_w
        L = POS2 // 2

        pltpu.sync_copy(pos2_hbm, pos_v)

        def chunk_body(c, carry):
            base = w_base + c * CH
            pltpu.sync_copy(tokens_hbm.at[pl.ds(base, CH)], tok_v)
            pltpu.sync_copy(segs_hbm.at[pl.ds(base, CH)], seg_v)
            gather = pltpu.async_copy(tok_table_hbm.at[tok_v], acc_v, sem)
            for j in range(CH // _L16):
                sl = pl.ds(j * _L16, _L16)
                t = tok_v[sl]
                s = seg_v[sl]
                cidx_v[sl] = jnp.where(t == 0, s + 2, s)
            gather.wait()
            pltpu.async_copy(aux_hbm.at[cidx_v], aux_v, sem).wait()

            p0 = lax.rem(c * CH, L)

            def row_body(r, carry2):
                pr = p0 + r
                for g in range(D // _L16):
                    sl = pl.ds(g * _L16, _L16)
                    acc_v[r, sl] = acc_v[r, sl] + aux_v[r, sl] + pos_v[pr, sl]
                return carry2

            lax.fori_loop(0, CH, row_body, 0, unroll=False)
            pltpu.sync_copy(acc_v, out_hbm.at[pl.ds(base, CH)])
            return carry

        lax.fori_loop(0, n_chunks, chunk_body, 0, unroll=False)

    return sc_kernel


def kernel(tokens, segments, tok_table, pos_table, seg_table):
    B, L = tokens.shape
    V, D = tok_table.shape
    N = B * L

    tok0 = tok_table[0]
    segz = seg_table.at[0].set(0.0)
    aux = jnp.concatenate([segz, segz - tok0[None, :]], axis=0)
    pos2 = jnp.concatenate([pos_table, pos_table], axis=0)

    tokens_f = tokens.reshape(-1).astype(jnp.int32)
    segs_f = segments.reshape(-1).astype(jnp.int32)

    sc = _make_sc_kernel(N, D, 128, 2 * L)
    out = sc(tokens_f, segs_f, tok_table, aux, pos2)
    return out.reshape(B, L, D)
